# Initial kernel scaffold; baseline (speedup 1.0000x reference)
import functools

import jax
import jax.numpy as jnp
from jax import lax
from jax.experimental import pallas as pl
from jax.experimental.pallas import tpu as pltpu
from jax.experimental.pallas import tpu_sc as plsc

N = 10000
E = 320000
D = 128
H = 128
C = 10
G = 64

NC = 2
NS = 16
NW = NC * NS
K = 128
NCHUNK = (E + NW * K - 1) // (NW * K)
E_PAD = NW * K * NCHUNK
EPT = NCHUNK * K
RPT = N // NS

_mesh = plsc.VectorSubcoreMesh(core_axis_name="c", subcore_axis_name="s")


@functools.partial(
    pl.kernel,
    out_type=jax.ShapeDtypeStruct((NC, N), jnp.float32),
    mesh=_mesh,
    scratch_types=[
        pltpu.VMEM((NCHUNK, K), jnp.int32),
        pltpu.VMEM((NCHUNK, K), jnp.float32),
        pltpu.VMEM_SHARED((N,), jnp.float32),
    ],
)
def _sc_wdeg(dstg, ewg, zeros_n, out, dst_v, ew_v, dacc):
    core = lax.axis_index("c")
    sid = lax.axis_index("s")
    wid = core * NS + sid

    pltpu.sync_copy(dstg.at[wid], dst_v)
    pltpu.sync_copy(ewg.at[wid], ew_v)

    @pl.when(sid < NS - 1)
    def _():
        pltpu.sync_copy(zeros_n.at[pl.ds(sid * 632, 632)],
                        dacc.at[pl.ds(sid * 632, 632)])

    @pl.when(sid == NS - 1)
    def _():
        pltpu.sync_copy(zeros_n.at[pl.ds(9480, 520)],
                        dacc.at[pl.ds(9480, 520)])

    plsc.subcore_barrier()

    def body(j, carry):
        pltpu.sync_copy(ew_v.at[j], dacc.at[dst_v.at[j]], add=True)
        return carry
    lax.fori_loop(0, NCHUNK, body, 0)

    plsc.subcore_barrier()

    @pl.when(sid < NS - 1)
    def _():
        pltpu.sync_copy(dacc.at[pl.ds(sid * 632, 632)],
                        out.at[core, pl.ds(sid * 632, 632)])

    @pl.when(sid == NS - 1)
    def _():
        pltpu.sync_copy(dacc.at[pl.ds(9480, 520)],
                        out.at[core, pl.ds(9480, 520)])


@functools.partial(
    pl.kernel,
    out_type=jax.ShapeDtypeStruct((NC, N, H), jnp.float32),
    mesh=_mesh,
    scratch_types=[
        pltpu.VMEM((NCHUNK, K), jnp.int32),
        pltpu.VMEM((NCHUNK, K), jnp.int32),
        pltpu.VMEM((NCHUNK, K), jnp.float32),
        pltpu.VMEM((K, H), jnp.float32),
        pltpu.VMEM_SHARED((N, H), jnp.float32),
        pltpu.SemaphoreType.DMA,
    ],
)
def _sc_agg(ys, srcg, dstg, ewg, zeros_nd, out,
            src_v, dst_v, ew_v, gbuf, acc, sem):
    core = lax.axis_index("c")
    sid = lax.axis_index("s")
    wid = core * NS + sid

    pltpu.sync_copy(srcg.at[wid], src_v)
    pltpu.sync_copy(dstg.at[wid], dst_v)
    pltpu.sync_copy(ewg.at[wid], ew_v)

    pltpu.sync_copy(zeros_nd.at[pl.ds(sid * RPT, RPT)],
                    acc.at[pl.ds(sid * RPT, RPT)])
    plsc.subcore_barrier()

    lanes = lax.iota(jnp.int32, NS)

    def chunk_body(j, carry):
        pltpu.async_copy(ys.at[src_v.at[j]], gbuf, sem).wait()

        def group_body(g, c2):
            rows = g * NS + lanes
            ewv = ew_v[j, pl.ds(g * NS, NS)]
            for c in range(H):
                colv = jnp.full((NS,), c, jnp.int32)
                v = plsc.load_gather(gbuf, [rows, colv])
                plsc.store_scatter(gbuf, [rows, colv], v * ewv)
            return c2
        lax.fori_loop(0, K // NS, group_body, 0)

        pltpu.sync_copy(gbuf, acc.at[dst_v.at[j]], add=True)
        return carry
    lax.fori_loop(0, NCHUNK, chunk_body, 0)

    plsc.subcore_barrier()
    pltpu.sync_copy(acc.at[pl.ds(sid * RPT, RPT)],
                    out.at[core, pl.ds(sid * RPT, RPT)])


def _tc_prep_body(x_ref, w_ref, wdeg_ref, ys_ref, dinv_ref):
    deg = 1.0 + wdeg_ref[:, 0:1] + wdeg_ref[:, 1:2]
    dinv = lax.rsqrt(deg)
    xw = jnp.dot(x_ref[...], w_ref[...],
                 preferred_element_type=jnp.float32,
                 precision=lax.Precision.HIGHEST)
    ys_ref[...] = xw * dinv
    dinv_ref[...] = dinv


def _tc_mid_body(acc_ref, ys_ref, dinv_ref, b_ref, w_ref, out_ref):
    dinv = dinv_ref[...]
    a = acc_ref[0] + acc_ref[1] + ys_ref[...]
    h = jnp.maximum(a * dinv + b_ref[...], 0.0)
    hw = jnp.dot(h, w_ref[...],
                 preferred_element_type=jnp.float32,
                 precision=lax.Precision.HIGHEST)
    out_ref[...] = hw * dinv


def _tc_final_body(acc_ref, ys_ref, dinv_ref, b_ref, batch_ref, wl_ref,
                   bl_ref, hg_ref, lp_ref):
    dinv = dinv_ref[...]
    h = (acc_ref[0] + acc_ref[1] + ys_ref[...]) * dinv + b_ref[...]
    seg = batch_ref[...]
    oh = (lax.broadcasted_iota(jnp.int32, (N, G), 1) == seg)
    hg = lax.dot_general(oh.astype(jnp.float32), h,
                         (((0,), (0,)), ((), ())),
                         preferred_element_type=jnp.float32,
                         precision=lax.Precision.HIGHEST)
    logits = jnp.dot(hg, wl_ref[...],
                     preferred_element_type=jnp.float32,
                     precision=lax.Precision.HIGHEST) + bl_ref[...]
    m = jnp.max(logits, axis=1, keepdims=True)
    lse = m + jnp.log(jnp.sum(jnp.exp(logits - m), axis=1, keepdims=True))
    hg_ref[...] = hg
    lp_ref[...] = logits - lse


_tc_prep = pl.pallas_call(
    _tc_prep_body,
    out_shape=(jax.ShapeDtypeStruct((N, H), jnp.float32),
               jax.ShapeDtypeStruct((N, 1), jnp.float32)),
)

_tc_mid = pl.pallas_call(
    _tc_mid_body,
    out_shape=jax.ShapeDtypeStruct((N, H), jnp.float32),
)

_tc_final = pl.pallas_call(
    _tc_final_body,
    out_shape=(jax.ShapeDtypeStruct((G, H), jnp.float32),
               jax.ShapeDtypeStruct((G, C), jnp.float32)),
)


def kernel(x, edge_index, batch, edge_weight, W1, b1, W2, b2, W3, b3, Wl, bl):
    src = edge_index[0]
    dst = edge_index[1]
    pad = E_PAD - E
    srcg = jnp.concatenate([src, jnp.zeros((pad,), src.dtype)]).reshape(NW, NCHUNK, K)
    dstg = jnp.concatenate([dst, jnp.zeros((pad,), dst.dtype)]).reshape(NW, NCHUNK, K)
    ewg = jnp.concatenate([edge_weight, jnp.zeros((pad,), edge_weight.dtype)]
                          ).reshape(NW, NCHUNK, K)
    zeros_n = jnp.zeros((N,), jnp.float32)
    zeros_nd = jnp.zeros((N, H), jnp.float32)
    b1r = b1.reshape(1, H)
    b2r = b2.reshape(1, H)
    b3r = b3.reshape(1, H)
    blr = bl.reshape(1, C)
    batchc = batch.reshape(N, 1)

    wdeg = _sc_wdeg(dstg, ewg, zeros_n)
    ys1, dinv = _tc_prep(x, W1, wdeg.T)
    acc1 = _sc_agg(ys1, srcg, dstg, ewg, zeros_nd)
    ys2 = _tc_mid(acc1, ys1, dinv, b1r, W2)
    acc2 = _sc_agg(ys2, srcg, dstg, ewg, zeros_nd)
    ys3 = _tc_mid(acc2, ys2, dinv, b2r, W3)
    acc3 = _sc_agg(ys3, srcg, dstg, ewg, zeros_nd)
    hG, logp = _tc_final(acc3, ys3, dinv, b3r, batchc, Wl, blr)
    return (hG, logp)


# speedup vs baseline: 2.0920x; 2.0920x over previous
import functools

import jax
import jax.numpy as jnp
from jax import lax
from jax.experimental import pallas as pl
from jax.experimental.pallas import tpu as pltpu
from jax.experimental.pallas import tpu_sc as plsc

N = 10000
E = 320000
D = 128
H = 128
C = 10
G = 64

NC = 2
NS = 16
NW = NC * NS
K = 128
NCHUNK = (E + NW * K - 1) // (NW * K)
E_PAD = NW * K * NCHUNK
EPT = NCHUNK * K
RPT_A = 632
RPT_B = N - (NS - 1) * RPT_A

_mesh = plsc.VectorSubcoreMesh(core_axis_name="c", subcore_axis_name="s")


N_PAD = 10240
RED = N_PAD // NS


@functools.partial(
    pl.kernel,
    out_type=jax.ShapeDtypeStruct((NC * N_PAD,), jnp.float32),
    mesh=_mesh,
    scratch_types=[
        pltpu.VMEM((NCHUNK, K), jnp.int32),
        pltpu.VMEM((NCHUNK, K), jnp.float32),
        pltpu.VMEM((N_PAD,), jnp.float32),
        pltpu.VMEM((RED,), jnp.float32),
        pltpu.VMEM((RED,), jnp.float32),
        pltpu.VMEM_SHARED((NS, N_PAD), jnp.float32),
    ],
    compiler_params=pltpu.CompilerParams(needs_layout_passes=False),
)
def _sc_wdeg(dstg, ewg, out, dst_v, ew_v, hist, racc, rtmp, shist):
    core = lax.axis_index("c")
    sid = lax.axis_index("s")
    wid = core * NS + sid

    pltpu.sync_copy(dstg.at[wid], dst_v)
    pltpu.sync_copy(ewg.at[wid], ew_v)

    zv = jnp.zeros((NS,), jnp.float32)

    def zero_body(i, carry):
        hist[pl.ds(i * NS, NS)] = zv
        return carry
    lax.fori_loop(0, N_PAD // NS, zero_body, 0)

    def chunk_body(j, carry):
        def group_body(g, c2):
            idx = dst_v[j, pl.ds(g * NS, NS)]
            w = ew_v[j, pl.ds(g * NS, NS)]
            plsc.addupdate_scatter(hist, [idx], w)
            return c2
        lax.fori_loop(0, K // NS, group_body, 0)
        return carry
    lax.fori_loop(0, NCHUNK, chunk_body, 0)

    pltpu.sync_copy(hist, shist.at[sid])
    plsc.subcore_barrier()

    def racc_zero(i, carry):
        racc[pl.ds(i * NS, NS)] = zv
        return carry
    lax.fori_loop(0, RED // NS, racc_zero, 0)

    def red_body(s, carry):
        pltpu.sync_copy(shist.at[s, pl.ds(sid * RED, RED)], rtmp)
        def add_body(i, c2):
            racc[pl.ds(i * NS, NS)] += rtmp[pl.ds(i * NS, NS)]
            return c2
        lax.fori_loop(0, RED // NS, add_body, 0)
        return carry
    lax.fori_loop(0, NS, red_body, 0)

    pltpu.sync_copy(racc, out.at[pl.ds(core * N_PAD + sid * RED, RED)])


@functools.partial(
    pl.kernel,
    out_type=jax.ShapeDtypeStruct((NC, N, H), jnp.float32),
    mesh=_mesh,
    scratch_types=[
        pltpu.VMEM((NCHUNK, K), jnp.int32),
        pltpu.VMEM((NCHUNK, K), jnp.int32),
        pltpu.VMEM((NCHUNK, K), jnp.float32),
        pltpu.VMEM((K, H), jnp.float32),
        pltpu.VMEM_SHARED((N, H), jnp.float32),
        pltpu.SemaphoreType.DMA,
    ],
    compiler_params=pltpu.CompilerParams(needs_layout_passes=False),
)
def _sc_agg(ys, srcg, dstg, ewg, zeros_nd, out,
            src_v, dst_v, ew_v, gbuf, acc, sem):
    core = lax.axis_index("c")
    sid = lax.axis_index("s")
    wid = core * NS + sid

    pltpu.sync_copy(srcg.at[wid], src_v)
    pltpu.sync_copy(dstg.at[wid], dst_v)
    pltpu.sync_copy(ewg.at[wid], ew_v)

    @pl.when(sid < NS - 1)
    def _():
        pltpu.sync_copy(zeros_nd.at[pl.ds(sid * RPT_A, RPT_A)],
                        acc.at[pl.ds(sid * RPT_A, RPT_A)])

    @pl.when(sid == NS - 1)
    def _():
        pltpu.sync_copy(zeros_nd.at[pl.ds((NS - 1) * RPT_A, RPT_B)],
                        acc.at[pl.ds((NS - 1) * RPT_A, RPT_B)])

    plsc.subcore_barrier()

    lanes = lax.iota(jnp.int32, NS)

    def chunk_body(j, carry):
        pltpu.async_copy(ys.at[src_v.at[j]], gbuf, sem).wait()

        def group_body(g, c2):
            rows = g * NS + lanes
            ewv = ew_v[j, pl.ds(g * NS, NS)]
            for c in range(H):
                colv = jnp.full((NS,), c, jnp.int32)
                v = plsc.load_gather(gbuf, [rows, colv])
                plsc.store_scatter(gbuf, [rows, colv], v * ewv)
            return c2
        lax.fori_loop(0, K // NS, group_body, 0)

        pltpu.sync_copy(gbuf, acc.at[dst_v.at[j]], add=True)
        return carry
    lax.fori_loop(0, NCHUNK, chunk_body, 0)

    plsc.subcore_barrier()

    @pl.when(sid < NS - 1)
    def _():
        pltpu.sync_copy(acc.at[pl.ds(sid * RPT_A, RPT_A)],
                        out.at[core, pl.ds(sid * RPT_A, RPT_A)])

    @pl.when(sid == NS - 1)
    def _():
        pltpu.sync_copy(acc.at[pl.ds((NS - 1) * RPT_A, RPT_B)],
                        out.at[core, pl.ds((NS - 1) * RPT_A, RPT_B)])


def _tc_prep_body(x_ref, w_ref, wdeg_ref, ys_ref, dinv_ref):
    deg = 1.0 + wdeg_ref[:, 0:1] + wdeg_ref[:, 1:2]
    dinv = lax.rsqrt(deg)
    xw = jnp.dot(x_ref[...], w_ref[...],
                 preferred_element_type=jnp.float32,
                 precision=lax.Precision.HIGHEST)
    ys_ref[...] = xw * dinv
    dinv_ref[...] = dinv


def _tc_mid_body(acc_ref, ys_ref, dinv_ref, b_ref, w_ref, out_ref):
    dinv = dinv_ref[...]
    a = acc_ref[0] + acc_ref[1] + ys_ref[...]
    h = jnp.maximum(a * dinv + b_ref[...], 0.0)
    hw = jnp.dot(h, w_ref[...],
                 preferred_element_type=jnp.float32,
                 precision=lax.Precision.HIGHEST)
    out_ref[...] = hw * dinv


def _tc_final_body(acc_ref, ys_ref, dinv_ref, b_ref, batch_ref, wl_ref,
                   bl_ref, hg_ref, lp_ref):
    dinv = dinv_ref[...]
    h = (acc_ref[0] + acc_ref[1] + ys_ref[...]) * dinv + b_ref[...]
    seg = batch_ref[...]
    oh = (lax.broadcasted_iota(jnp.int32, (N, G), 1) == seg)
    hg = lax.dot_general(oh.astype(jnp.float32), h,
                         (((0,), (0,)), ((), ())),
                         preferred_element_type=jnp.float32,
                         precision=lax.Precision.HIGHEST)
    logits = jnp.dot(hg, wl_ref[...],
                     preferred_element_type=jnp.float32,
                     precision=lax.Precision.HIGHEST) + bl_ref[...]
    m = jnp.max(logits, axis=1, keepdims=True)
    lse = m + jnp.log(jnp.sum(jnp.exp(logits - m), axis=1, keepdims=True))
    hg_ref[...] = hg
    lp_ref[...] = logits - lse


_tc_prep = pl.pallas_call(
    _tc_prep_body,
    out_shape=(jax.ShapeDtypeStruct((N, H), jnp.float32),
               jax.ShapeDtypeStruct((N, 1), jnp.float32)),
)

_tc_mid = pl.pallas_call(
    _tc_mid_body,
    out_shape=jax.ShapeDtypeStruct((N, H), jnp.float32),
)

_tc_final = pl.pallas_call(
    _tc_final_body,
    out_shape=(jax.ShapeDtypeStruct((G, H), jnp.float32),
               jax.ShapeDtypeStruct((G, C), jnp.float32)),
)


def kernel(x, edge_index, batch, edge_weight, W1, b1, W2, b2, W3, b3, Wl, bl):
    src = edge_index[0]
    dst = edge_index[1]
    pad = E_PAD - E
    srcg = jnp.concatenate([src, jnp.zeros((pad,), src.dtype)]).reshape(NW, NCHUNK, K)
    dstg = jnp.concatenate([dst, jnp.zeros((pad,), dst.dtype)]).reshape(NW, NCHUNK, K)
    ewg = jnp.concatenate([edge_weight, jnp.zeros((pad,), edge_weight.dtype)]
                          ).reshape(NW, NCHUNK, K)
    zeros_nd = jnp.zeros((N, H), jnp.float32)
    b1r = b1.reshape(1, H)
    b2r = b2.reshape(1, H)
    b3r = b3.reshape(1, H)
    blr = bl.reshape(1, C)
    batchc = batch.reshape(N, 1)

    wdeg = _sc_wdeg(dstg, ewg).reshape(NC, N_PAD)[:, :N]
    ys1, dinv = _tc_prep(x, W1, wdeg.T)
    acc1 = _sc_agg(ys1, srcg, dstg, ewg, zeros_nd)
    ys2 = _tc_mid(acc1, ys1, dinv, b1r, W2)
    acc2 = _sc_agg(ys2, srcg, dstg, ewg, zeros_nd)
    ys3 = _tc_mid(acc2, ys2, dinv, b2r, W3)
    acc3 = _sc_agg(ys3, srcg, dstg, ewg, zeros_nd)
    hG, logp = _tc_final(acc3, ys3, dinv, b3r, batchc, Wl, blr)
    return (hG, logp)
